# baseline (speedup 1.0000x reference)
import functools

import jax
import jax.numpy as jnp
from jax import lax
from jax.experimental import pallas as pl
from jax.experimental.pallas import tpu as pltpu
from jax.experimental.pallas import tpu_sc as plsc


def _ln_rows(x):
    m = jnp.mean(x, axis=-1, keepdims=True)
    v = jnp.mean((x - m) ** 2, axis=-1, keepdims=True)
    return (x - m) / jnp.sqrt(v + 1e-5)


def _text_body(text_ref, wt_ref, bt_ref, gate_ref, out_ref):
    t = lax.dot_general(text_ref[...], wt_ref[...],
                        (((1,), (1,)), ((), ())),
                        preferred_element_type=jnp.float32)
    t = jax.nn.relu(t + bt_ref[...]) * 0.1
    tn = _ln_rows(t)
    gate = jax.nn.sigmoid(gate_ref[0, 0])
    out_ref[...] = gate * tn


def _mlp_body(sp_ref, ws_ref, bs_ref, tg_ref, wo_ref, bo_ref, fused_ref):
    sp = sp_ref[0]
    fs = lax.dot_general(sp, ws_ref[...], (((1,), (1,)), ((), ())),
                         preferred_element_type=jnp.float32) + bs_ref[...]
    fsn = _ln_rows(fs) + tg_ref[0]
    fo = lax.dot_general(fsn, wo_ref[...], (((1,), (1,)), ((), ())),
                         preferred_element_type=jnp.float32) + bo_ref[...]
    nrm = jnp.sqrt(jnp.sum(fo * fo, axis=1, keepdims=True))
    spn = jnp.sqrt(jnp.sum(sp * sp, axis=1, keepdims=True))
    fused_ref[0] = fo / jnp.maximum(nrm, 1e-12) * spn


def _topk_body(d_ref, idx_ref):
    R, L = d_ref.shape
    v = d_ref[...]
    b = lax.bitcast_convert_type(v, jnp.int32)
    s = jnp.where(b < 0, b ^ jnp.int32(0x7FFFFFFF), b)
    row = lax.broadcasted_iota(jnp.int32, (R, L), 0)
    col = lax.broadcasted_iota(jnp.int32, (R, L), 1)
    ilin = ((row & 7) << 9) + col
    idx = ilin
    for lk in range(1, 13):
        k = 1 << lk
        A = (ilin & k) == 0
        for lj in range(lk - 1, -1, -1):
            d = 1 << lj
            Lm = (ilin & d) == 0
            if d < L:
                s_up, s_dn = jnp.roll(s, -d, axis=1), jnp.roll(s, d, axis=1)
                i_up, i_dn = jnp.roll(idx, -d, axis=1), jnp.roll(idx, d, axis=1)
            else:
                rd = d >> 9
                s_up, s_dn = jnp.roll(s, -rd, axis=0), jnp.roll(s, rd, axis=0)
                i_up, i_dn = jnp.roll(idx, -rd, axis=0), jnp.roll(idx, rd, axis=0)
            ps = jnp.where(Lm, s_up, s_dn)
            pi = jnp.where(Lm, i_up, i_dn)
            g = (s > ps) | ((s == ps) & (idx < pi))
            take_self = g == (Lm == A)
            s = jnp.where(take_self, s, ps)
            idx = jnp.where(take_self, idx, pi)
    idx_ref[...] = idx


def _make_sc_gather(n_rows, C):
    mesh = plsc.VectorSubcoreMesh(core_axis_name="c", subcore_axis_name="s")
    per_w = n_rows // 32

    @functools.partial(
        pl.kernel, mesh=mesh,
        out_type=jax.ShapeDtypeStruct((n_rows, C), jnp.float32),
        scratch_types=[
            pltpu.VMEM((128,), jnp.int32),
            pltpu.VMEM((128, C), jnp.float32),
            pltpu.SemaphoreType.DMA,
        ],
    )
    def _sc_gather(table_hbm, idx_hbm, out_hbm, idx_v, rows_v, sem):
        wid = lax.axis_index("s") * 2 + lax.axis_index("c")
        for j in range(per_w // 128):
            base = wid * per_w + j * 128
            pltpu.sync_copy(idx_hbm.at[pl.ds(base, 128)], idx_v)
            pltpu.async_copy(table_hbm.at[idx_v], rows_v, sem).wait()
            pltpu.sync_copy(rows_v, out_hbm.at[pl.ds(base, 128)])

    return _sc_gather


def _scatter_body(t_ref, fused_ref, idx_ref, out_ref, ft_scr):
    strip = pl.program_id(1)
    SW = out_ref.shape[2]

    @pl.when(strip == 0)
    def _():
        ft_scr[...] = lax.transpose(fused_ref[0], (1, 0)).astype(jnp.bfloat16)

    idxT = lax.transpose(idx_ref[0], (1, 0))
    p = lax.broadcasted_iota(jnp.int32, (512, SW), 1) + strip * SW
    acc = jnp.zeros((out_ref.shape[1], SW), jnp.float32)
    msk = jnp.zeros((512, SW), jnp.float32)
    for c in range(4):
        oh = (idxT[:, c:c + 1] == p)
        ohb = oh.astype(jnp.bfloat16)
        msk = msk + ohb.astype(jnp.float32)
        acc = acc + lax.dot_general(
            ft_scr[:, c * 512:(c + 1) * 512], ohb,
            (((1,), (0,)), ((), ())), preferred_element_type=jnp.float32)
    sel = jnp.sum(msk, axis=0, keepdims=True) > 0.0
    out_ref[0] = jnp.where(sel, acc, t_ref[0])


def kernel(tensor, text_emb, W1, b1, W2, b2, Ws, bs, Wt, bt, Wo, bo, gate_param):
    B, C, H, Wd = tensor.shape
    HW = H * Wd
    K = max(1, int(HW * 0.5))
    embed_dim = Ws.shape[0]

    t3 = tensor.reshape(B, C, HW)
    feat_flat = jnp.transpose(t3, (0, 2, 1))
    dn = ('NCHW', 'OIHW', 'NCHW')
    x = lax.conv_general_dilated(tensor, W1, (1, 1), [(0, 0), (0, 0)],
                                 dimension_numbers=dn) + b1.reshape(1, -1, 1, 1)
    x1 = lax.conv_general_dilated(x, W2, (1, 1), [(2, 2), (2, 2)],
                                  rhs_dilation=(2, 2), dimension_numbers=dn) + b2.reshape(1, -1, 1, 1)
    density_map = jax.nn.relu(x1 + x)
    idx_sorted = pl.pallas_call(
        _topk_body,
        out_shape=jax.ShapeDtypeStruct((B * 8, HW // 8), jnp.int32),
    )(density_map.reshape(B * 8, HW // 8))
    idx_b = idx_sorted.reshape(B, 8, HW // 8)
    topk_idx = idx_b[:, :K // (HW // 8), :].reshape(B, K)
    offs = (jnp.arange(B, dtype=jnp.int32) * HW)[:, None]
    sel_flat = (topk_idx + offs).reshape(-1)
    table = feat_flat.reshape(B * HW, C)
    sparse_feat = _make_sc_gather(B * K, C)(table, sel_flat).reshape(B, K, C)

    tg = pl.pallas_call(
        _text_body,
        out_shape=jax.ShapeDtypeStruct((B, embed_dim), jnp.float32),
    )(text_emb, Wt, bt.reshape(1, -1), gate_param.reshape(1, 1))

    RB = 512
    fused = pl.pallas_call(
        _mlp_body,
        grid=(B, K // RB),
        in_specs=[
            pl.BlockSpec((1, RB, C), lambda b, r: (b, r, 0)),
            pl.BlockSpec((embed_dim, C), lambda b, r: (0, 0)),
            pl.BlockSpec((1, embed_dim), lambda b, r: (0, 0)),
            pl.BlockSpec((1, 1, embed_dim), lambda b, r: (b, 0, 0)),
            pl.BlockSpec((C, embed_dim), lambda b, r: (0, 0)),
            pl.BlockSpec((1, C), lambda b, r: (0, 0)),
        ],
        out_specs=pl.BlockSpec((1, RB, C), lambda b, r: (b, r, 0)),
        out_shape=jax.ShapeDtypeStruct((B, K, C), jnp.float32),
    )(sparse_feat, Ws, bs.reshape(1, -1), tg.reshape(B, 1, embed_dim), Wo, bo.reshape(1, -1))

    SW = HW // 4
    out = pl.pallas_call(
        _scatter_body,
        grid=(B, 4),
        in_specs=[
            pl.BlockSpec((1, C, SW), lambda b, s: (b, 0, s)),
            pl.BlockSpec((1, K, C), lambda b, s: (b, 0, 0)),
            pl.BlockSpec((1, 8, HW // 8), lambda b, s: (b, 0, 0)),
        ],
        out_specs=pl.BlockSpec((1, C, SW), lambda b, s: (b, 0, s)),
        out_shape=jax.ShapeDtypeStruct((B, C, HW), jnp.float32),
        scratch_shapes=[pltpu.VMEM((C, K), jnp.bfloat16)],
    )(t3, fused, idx_b)
    out = out.reshape(B, C, H, Wd)
    return out, density_map, topk_idx, fused


# speedup vs baseline: 1.5830x; 1.5830x over previous
import functools

import jax
import jax.numpy as jnp
from jax import lax
from jax.experimental import pallas as pl
from jax.experimental.pallas import tpu as pltpu
from jax.experimental.pallas import tpu_sc as plsc


def _ln_rows(x):
    m = jnp.mean(x, axis=-1, keepdims=True)
    v = jnp.mean((x - m) ** 2, axis=-1, keepdims=True)
    return (x - m) / jnp.sqrt(v + 1e-5)


def _text_body(text_ref, wt_ref, bt_ref, gate_ref, out_ref):
    t = lax.dot_general(text_ref[...], wt_ref[...],
                        (((1,), (1,)), ((), ())),
                        preferred_element_type=jnp.float32)
    t = jax.nn.relu(t + bt_ref[...]) * 0.1
    tn = _ln_rows(t)
    gate = jax.nn.sigmoid(gate_ref[0, 0])
    out_ref[...] = gate * tn


def _mlp_body(sp_ref, ws_ref, bs_ref, tg_ref, wo_ref, bo_ref, fused_ref):
    sp = sp_ref[0]
    fs = lax.dot_general(sp, ws_ref[...], (((1,), (1,)), ((), ())),
                         preferred_element_type=jnp.float32) + bs_ref[...]
    fsn = _ln_rows(fs) + tg_ref[0]
    fo = lax.dot_general(fsn, wo_ref[...], (((1,), (1,)), ((), ())),
                         preferred_element_type=jnp.float32) + bo_ref[...]
    nrm = jnp.sqrt(jnp.sum(fo * fo, axis=1, keepdims=True))
    spn = jnp.sqrt(jnp.sum(sp * sp, axis=1, keepdims=True))
    fused_ref[0] = fo / jnp.maximum(nrm, 1e-12) * spn


def _topk_body(d_ref, idx_ref):
    R, L = d_ref.shape
    v = d_ref[...]
    b = lax.bitcast_convert_type(v, jnp.int32)
    s = jnp.where(b < 0, b ^ jnp.int32(0x7FFFFFFF), b)
    row = lax.broadcasted_iota(jnp.int32, (R, L), 0)
    col = lax.broadcasted_iota(jnp.int32, (R, L), 1)
    ilin = ((row & 7) << 9) + col
    idx = ilin
    for lk in range(1, 13):
        k = 1 << lk
        A = (ilin & k) == 0
        for lj in range(lk - 1, -1, -1):
            d = 1 << lj
            Lm = (ilin & d) == 0
            if d < L:
                s_up, s_dn = jnp.roll(s, -d, axis=1), jnp.roll(s, d, axis=1)
                i_up, i_dn = jnp.roll(idx, -d, axis=1), jnp.roll(idx, d, axis=1)
            else:
                rd = d >> 9
                s_up, s_dn = jnp.roll(s, -rd, axis=0), jnp.roll(s, rd, axis=0)
                i_up, i_dn = jnp.roll(idx, -rd, axis=0), jnp.roll(idx, rd, axis=0)
            ps = jnp.where(Lm, s_up, s_dn)
            pi = jnp.where(Lm, i_up, i_dn)
            g = (s > ps) | ((s == ps) & (idx < pi))
            take_self = g == (Lm == A)
            s = jnp.where(take_self, s, ps)
            idx = jnp.where(take_self, idx, pi)
    idx_ref[...] = idx


def _make_sc_gather(n_rows, C):
    mesh = plsc.VectorSubcoreMesh(core_axis_name="c", subcore_axis_name="s")
    per_w = n_rows // 32

    @functools.partial(
        pl.kernel, mesh=mesh,
        out_type=jax.ShapeDtypeStruct((n_rows, C), jnp.float32),
        scratch_types=[
            pltpu.VMEM((128,), jnp.int32),
            pltpu.VMEM((128, C), jnp.float32),
            pltpu.SemaphoreType.DMA,
        ],
    )
    def _sc_gather(table_hbm, idx_hbm, out_hbm, idx_v, rows_v, sem):
        wid = lax.axis_index("s") * 2 + lax.axis_index("c")
        for j in range(per_w // 128):
            base = wid * per_w + j * 128
            pltpu.sync_copy(idx_hbm.at[pl.ds(base, 128)], idx_v)
            pltpu.async_copy(table_hbm.at[idx_v], rows_v, sem).wait()
            pltpu.sync_copy(rows_v, out_hbm.at[pl.ds(base, 128)])

    return _sc_gather


def _make_sc_scatter(n_total, n_sel, C):
    mesh = plsc.VectorSubcoreMesh(core_axis_name="c", subcore_axis_name="s")
    per_w = n_sel // 32

    @functools.partial(
        pl.kernel, mesh=mesh,
        out_type=jax.ShapeDtypeStruct((n_total, C), jnp.float32),
        scratch_types=[
            pltpu.VMEM((128,), jnp.int32),
            pltpu.VMEM((128, C), jnp.float32),
            pltpu.SemaphoreType.DMA,
        ],
    )
    def _sc_scatter(feat_hbm, fused_hbm, sel_hbm, unsel_hbm, out_hbm,
                    idx_v, rows_v, sem):
        wid = lax.axis_index("s") * 2 + lax.axis_index("c")
        for j in range(per_w // 128):
            base = wid * per_w + j * 128
            pltpu.sync_copy(unsel_hbm.at[pl.ds(base, 128)], idx_v)
            pltpu.async_copy(feat_hbm.at[idx_v], rows_v, sem).wait()
            pltpu.async_copy(rows_v, out_hbm.at[idx_v], sem).wait()
            pltpu.sync_copy(sel_hbm.at[pl.ds(base, 128)], idx_v)
            pltpu.sync_copy(fused_hbm.at[pl.ds(base, 128)], rows_v)
            pltpu.async_copy(rows_v, out_hbm.at[idx_v], sem).wait()

    return _sc_scatter


def kernel(tensor, text_emb, W1, b1, W2, b2, Ws, bs, Wt, bt, Wo, bo, gate_param):
    B, C, H, Wd = tensor.shape
    HW = H * Wd
    K = max(1, int(HW * 0.5))
    embed_dim = Ws.shape[0]

    t3 = tensor.reshape(B, C, HW)
    feat_flat = jnp.transpose(t3, (0, 2, 1))
    dn = ('NCHW', 'OIHW', 'NCHW')
    x = lax.conv_general_dilated(tensor, W1, (1, 1), [(0, 0), (0, 0)],
                                 dimension_numbers=dn) + b1.reshape(1, -1, 1, 1)
    x1 = lax.conv_general_dilated(x, W2, (1, 1), [(2, 2), (2, 2)],
                                  rhs_dilation=(2, 2), dimension_numbers=dn) + b2.reshape(1, -1, 1, 1)
    density_map = jax.nn.relu(x1 + x)
    idx_sorted = pl.pallas_call(
        _topk_body,
        out_shape=jax.ShapeDtypeStruct((B * 8, HW // 8), jnp.int32),
    )(density_map.reshape(B * 8, HW // 8))
    idx_b = idx_sorted.reshape(B, 8, HW // 8)
    topk_idx = idx_b[:, :K // (HW // 8), :].reshape(B, K)
    offs = (jnp.arange(B, dtype=jnp.int32) * HW)[:, None]
    sel_flat = (topk_idx + offs).reshape(-1)
    unsel_flat = (idx_b[:, K // (HW // 8):, :].reshape(B, HW - K) + offs).reshape(-1)
    table = feat_flat.reshape(B * HW, C)
    sparse_feat = _make_sc_gather(B * K, C)(table, sel_flat).reshape(B, K, C)

    tg = pl.pallas_call(
        _text_body,
        out_shape=jax.ShapeDtypeStruct((B, embed_dim), jnp.float32),
    )(text_emb, Wt, bt.reshape(1, -1), gate_param.reshape(1, 1))

    RB = 512
    fused = pl.pallas_call(
        _mlp_body,
        grid=(B, K // RB),
        in_specs=[
            pl.BlockSpec((1, RB, C), lambda b, r: (b, r, 0)),
            pl.BlockSpec((embed_dim, C), lambda b, r: (0, 0)),
            pl.BlockSpec((1, embed_dim), lambda b, r: (0, 0)),
            pl.BlockSpec((1, 1, embed_dim), lambda b, r: (b, 0, 0)),
            pl.BlockSpec((C, embed_dim), lambda b, r: (0, 0)),
            pl.BlockSpec((1, C), lambda b, r: (0, 0)),
        ],
        out_specs=pl.BlockSpec((1, RB, C), lambda b, r: (b, r, 0)),
        out_shape=jax.ShapeDtypeStruct((B, K, C), jnp.float32),
    )(sparse_feat, Ws, bs.reshape(1, -1), tg.reshape(B, 1, embed_dim), Wo, bo.reshape(1, -1))

    out_flat = _make_sc_scatter(B * HW, B * K, C)(
        table, fused.reshape(B * K, C), sel_flat, unsel_flat)
    out = jnp.transpose(out_flat.reshape(B, HW, C), (0, 2, 1)).reshape(B, C, H, Wd)
    return out, density_map, topk_idx, fused
